# baseline (speedup 1.0000x reference)
import functools

import jax
import jax.numpy as jnp
from jax import lax
from jax.experimental import pallas as pl
from jax.experimental.pallas import tpu as pltpu

_RB = 64
_GRP = 16


def _stream_body(t_ref, x_ref, loss_ref, *, v_total, n_rows_blk):
    neg_inf = jnp.float32(-jnp.inf)
    lane = lax.broadcasted_iota(jnp.int32, (1, 128), 1)
    n_full = v_total // 128
    tail = v_total - n_full * 128
    n_grp = n_full // _GRP
    rest = n_full - n_grp * _GRP

    t = t_ref[...]
    zero = jnp.zeros((_RB, 128), jnp.float32)

    def chunk_cols(c):
        return pl.ds(128 * c, 128)

    def p1_chunk(c, m128, p128):
        xc = x_ref[:, chunk_cols(c)]
        m128 = jnp.maximum(m128, xc)
        hit = (t - 128 * c) == lane
        p128 = p128 + jnp.where(hit, xc, 0.0)
        return m128, p128

    def p1_group(g, carry):
        m128, p128 = carry
        for u in range(_GRP):
            m128, p128 = p1_chunk(g * _GRP + u, m128, p128)
        return m128, p128

    m128 = jnp.full((_RB, 128), neg_inf, jnp.float32)
    m128, p128 = lax.fori_loop(0, n_grp, p1_group, (m128, zero))
    for u in range(rest):
        m128, p128 = p1_chunk(n_grp * _GRP + u, m128, p128)
    xt = x_ref[:, pl.ds(128 * n_full, tail)]
    lane_t = lax.broadcasted_iota(jnp.int32, (1, tail), 1)
    m_row = jnp.max(m128, axis=1, keepdims=True)
    m_row = jnp.maximum(m_row, jnp.max(xt, axis=1, keepdims=True))
    hit_t = (t - 128 * n_full) == lane_t
    p_row = (jnp.sum(p128, axis=1, keepdims=True) +
             jnp.sum(jnp.where(hit_t, xt, 0.0), axis=1, keepdims=True))

    def p2_chunk(c, s128):
        xc = x_ref[:, chunk_cols(c)]
        return s128 + jnp.exp(xc - m_row)

    def p2_group(g, s128):
        for u in range(_GRP):
            s128 = p2_chunk(g * _GRP + u, s128)
        return s128

    s128 = lax.fori_loop(0, n_grp, p2_group, zero)
    for u in range(rest):
        s128 = p2_chunk(n_grp * _GRP + u, s128)
    s_row = jnp.sum(s128, axis=1, keepdims=True)
    s_row = s_row + jnp.sum(jnp.exp(xt - m_row), axis=1, keepdims=True)

    loss_ref[...] = m_row + jnp.log(s_row) - p_row


def _topk_body(loss_ref, out_ref, *, k):
    loss = loss_ref[...]
    b = lax.bitcast_convert_type(loss, jnp.int32)
    key = b ^ (lax.shift_right_arithmetic(b, 31) & jnp.int32(0x7FFFFFFF))

    def cnt_ge(thresh):
        return jnp.sum((key >= thresh).astype(jnp.int32))

    base0 = jnp.where(cnt_ge(jnp.int32(0)) >= k, jnp.int32(0),
                      jnp.int32(-(2**31)))

    def body(i, base):
        cand = base | lax.shift_left(jnp.int32(1), 30 - i)
        return jnp.where(cnt_ge(cand) >= k, cand, base)

    big_t = lax.fori_loop(0, 31, body, base0)
    tb = big_t ^ (lax.shift_right_arithmetic(big_t, 31) & jnp.int32(0x7FFFFFFF))
    tval = lax.bitcast_convert_type(tb, jnp.float32)
    gt = loss > tval
    cnt_gt = jnp.sum(gt.astype(jnp.float32))
    sum_gt = jnp.sum(jnp.where(gt, loss, 0.0))
    res = (sum_gt + (jnp.float32(k) - cnt_gt) * tval) / jnp.float32(k)
    out_ref[...] = jnp.full((1, 1), res, jnp.float32)


@jax.jit
def kernel(inputs, targets):
    n, v = inputs.shape
    k = int(0.25 * n)
    t2 = targets.reshape(n, 1).astype(jnp.int32)
    n_blk = n // _RB
    loss = pl.pallas_call(
        functools.partial(_stream_body, v_total=v, n_rows_blk=_RB),
        grid=(n_blk,),
        in_specs=[
            pl.BlockSpec((_RB, 1), lambda j: (j, 0)),
            pl.BlockSpec((_RB, v), lambda j: (j, 0)),
        ],
        out_specs=pl.BlockSpec((_RB, 1), lambda j: (j, 0)),
        out_shape=jax.ShapeDtypeStruct((n, 1), jnp.float32),
        compiler_params=pltpu.CompilerParams(
            dimension_semantics=("arbitrary",)),
    )(t2, inputs)
    loss8 = loss.reshape(8, n // 8)
    out = pl.pallas_call(
        functools.partial(_topk_body, k=k),
        out_shape=jax.ShapeDtypeStruct((1, 1), jnp.float32),
    )(loss8)
    return out[0, 0]
